# baseline (speedup 1.0000x reference)
import jax
import jax.numpy as jnp
from jax import lax
from jax.experimental import pallas as pl
from jax.experimental.pallas import tpu as pltpu
from jax.experimental.pallas import tpu_sc as plsc

H = 64
S = 200
B = 1024
N = B * S
EPS = 1e-12

NC = 2
NS = 16
NW = NC * NS
PER_W = N // NW
G = 64
NBLK = PER_W // G
L = 16
Q = H // L


def _rsqrt(x):
    xi = plsc.bitcast(x, jnp.int32)
    yi = jnp.int32(0x5F3759DF) - (xi >> 1)
    y = plsc.bitcast(yi, jnp.float32)
    xh = x * 0.5
    for _ in range(2):
        y = y * (1.5 - xh * y * y)
    return y


def _body(tok_hbm, sem_hbm, spat_hbm, gamma_hbm, beta_hbm, out_hbm,
          idx_v, spat_v, gamma_v, beta_v, rows_v, out_v, gsem):
    wid = lax.axis_index("s") * NC + lax.axis_index("c")
    row_base = wid * PER_W

    pltpu.sync_copy(tok_hbm.at[pl.ds(row_base, PER_W)], idx_v)
    pltpu.sync_copy(spat_hbm, spat_v)
    pltpu.sync_copy(gamma_hbm, gamma_v)
    pltpu.sync_copy(beta_hbm, beta_v)

    iota = lax.iota(jnp.int32, L)

    def blk(k, carry):
        g0 = k * G
        pltpu.async_copy(sem_hbm.at[idx_v.at[pl.ds(g0, G)]], rows_v, gsem).wait()

        def chunk(cc, c2):
            c0 = cc * L
            gq = [gamma_v[pl.ds(q * L, L)] for q in range(Q)]
            bq = [beta_v[pl.ds(q * L, L)] for q in range(Q)]
            for r in range(L):
                rr = c0 + r
                sb = jnp.remainder(g0 + rr, S) * H
                for q in range(Q):
                    out_v[pl.ds(rr * H + q * L, L)] = (
                        rows_v[rr, pl.ds(q * L, L)]
                        + spat_v[pl.ds(sb + q * L, L)])
            base_idx = (iota + c0) * H
            s = jnp.zeros((L,), jnp.float32)
            s2 = jnp.zeros((L,), jnp.float32)
            for h in range(H):
                col = plsc.load_gather(out_v, [base_idx + h])
                s = s + col
                s2 = s2 + col * col
            mean = s * (1.0 / H)
            var = s2 * (1.0 / H) - mean * mean
            rstd = _rsqrt(var + EPS)
            for r in range(L):
                rr = c0 + r
                m = jnp.full((L,), mean[r], jnp.float32)
                rs = jnp.full((L,), rstd[r], jnp.float32)
                for q in range(Q):
                    sl = pl.ds(rr * H + q * L, L)
                    out_v[sl] = (out_v[sl] - m) * (rs * gq[q]) + bq[q]
            return c2

        lax.fori_loop(0, G // L, chunk, 0)
        pltpu.sync_copy(out_v, out_hbm.at[pl.ds((row_base + g0) * H, G * H)])
        return carry

    lax.fori_loop(0, NBLK, blk, 0)


def kernel(token_idx, semantic_table, spatial_table, gamma, beta):
    tok1d = token_idx.reshape(N).astype(jnp.int32)
    spat = spatial_table[:S].reshape(S * H)
    mesh = plsc.VectorSubcoreMesh(core_axis_name="c", subcore_axis_name="s")
    f = pl.kernel(
        _body,
        out_type=jax.ShapeDtypeStruct((N * H,), jnp.float32),
        mesh=mesh,
        compiler_params=pltpu.CompilerParams(
            use_tc_tiling_on_sc=False, needs_layout_passes=False),
        scratch_types=[
            pltpu.VMEM((PER_W,), jnp.int32),
            pltpu.VMEM((S * H,), jnp.float32),
            pltpu.VMEM((H,), jnp.float32),
            pltpu.VMEM((H,), jnp.float32),
            pltpu.VMEM((G, H), jnp.float32),
            pltpu.VMEM((G * H,), jnp.float32),
            pltpu.SemaphoreType.DMA,
        ],
    )
    out = f(tok1d, semantic_table, spat, gamma, beta)
    return out.reshape(B, S, H)


# speedup vs baseline: 1.3950x; 1.3950x over previous
import jax
import jax.numpy as jnp
from jax import lax
from jax.experimental import pallas as pl
from jax.experimental.pallas import tpu as pltpu
from jax.experimental.pallas import tpu_sc as plsc

H = 64
S = 200
B = 1024
N = B * S
EPS = 1e-12

NC = 2
NS = 16
NW = NC * NS
PER_W = N // NW
G = 64
NBLK = PER_W // G
L = 16
Q = H // L


def _rsqrt(x):
    xi = plsc.bitcast(x, jnp.int32)
    yi = jnp.int32(0x5F3759DF) - (xi >> 1)
    y = plsc.bitcast(yi, jnp.float32)
    xh = x * 0.5
    for _ in range(2):
        y = y * (1.5 - xh * y * y)
    return y


def _body(tok_hbm, sem_hbm, spat_hbm, gamma_hbm, beta_hbm, out_hbm,
          idx_v, spat_v, gamma_v, beta_v,
          rows_a, rows_b, out_a, out_b, gsa, gsb, osa, osb):
    wid = lax.axis_index("s") * NC + lax.axis_index("c")
    row_base = wid * PER_W

    pltpu.sync_copy(tok_hbm.at[pl.ds(row_base, PER_W)], idx_v)
    pltpu.sync_copy(spat_hbm, spat_v)
    pltpu.sync_copy(gamma_hbm, gamma_v)
    pltpu.sync_copy(beta_hbm, beta_v)

    def g_src(k):
        return sem_hbm.at[idx_v.at[pl.ds(k * G, G)]]

    def o_dst(k):
        return out_hbm.at[pl.ds((row_base + k * G) * H, G * H)]

    def compute(k, rows_v, out_v):
        g0 = k * G

        def chunk(cc, c2):
            c0 = cc * L
            gq = [gamma_v[pl.ds(q * L, L)] for q in range(Q)]
            bq = [beta_v[pl.ds(q * L, L)] for q in range(Q)]
            for r in range(L):
                rr = c0 + r
                sb = jnp.remainder(g0 + rr, S) * H
                c = [rows_v[rr, pl.ds(q * L, L)] + spat_v[pl.ds(sb + q * L, L)]
                     for q in range(Q)]
                sv = (c[0] + c[1]) + (c[2] + c[3])
                s2 = ((c[0] * c[0] + c[1] * c[1])
                      + (c[2] * c[2] + c[3] * c[3]))
                tot = jnp.full((L,), lax.reduce_sum_p.bind(sv, axes=(0,)),
                               jnp.float32)
                tot2 = jnp.full((L,), lax.reduce_sum_p.bind(s2, axes=(0,)),
                                jnp.float32)
                mean = tot * (1.0 / H)
                var = tot2 * (1.0 / H) - mean * mean
                rstd = _rsqrt(var + EPS)
                ob = rr * H
                for q in range(Q):
                    o = (c[q] - mean) * (rstd * gq[q]) + bq[q]
                    out_v[pl.ds(ob + q * L, L)] = o
            return c2

        lax.fori_loop(0, G // L, chunk, 0)

    pltpu.async_copy(g_src(0), rows_a, gsa)

    def blk(kk, carry):
        k0 = 2 * kk
        k1 = k0 + 1
        pltpu.async_copy(g_src(k1), rows_b, gsb)
        pltpu.make_async_copy(g_src(k0), rows_a, gsa).wait()

        @pl.when(kk > 0)
        def _():
            pltpu.make_async_copy(out_a, o_dst(k0 - 2), osa).wait()

        compute(k0, rows_a, out_a)
        pltpu.async_copy(out_a, o_dst(k0), osa)

        @pl.when(kk + 1 < NBLK // 2)
        def _():
            pltpu.async_copy(g_src(k0 + 2), rows_a, gsa)

        pltpu.make_async_copy(g_src(k1), rows_b, gsb).wait()

        @pl.when(kk > 0)
        def _():
            pltpu.make_async_copy(out_b, o_dst(k1 - 2), osb).wait()

        compute(k1, rows_b, out_b)
        pltpu.async_copy(out_b, o_dst(k1), osb)
        return carry

    lax.fori_loop(0, NBLK // 2, blk, 0)
    pltpu.make_async_copy(out_a, o_dst(NBLK - 2), osa).wait()
    pltpu.make_async_copy(out_b, o_dst(NBLK - 1), osb).wait()


def kernel(token_idx, semantic_table, spatial_table, gamma, beta):
    tok1d = token_idx.reshape(N).astype(jnp.int32)
    spat = spatial_table[:S].reshape(S * H)
    mesh = plsc.VectorSubcoreMesh(core_axis_name="c", subcore_axis_name="s")
    f = pl.kernel(
        _body,
        out_type=jax.ShapeDtypeStruct((N * H,), jnp.float32),
        mesh=mesh,
        compiler_params=pltpu.CompilerParams(
            use_tc_tiling_on_sc=False, needs_layout_passes=False),
        scratch_types=[
            pltpu.VMEM((PER_W,), jnp.int32),
            pltpu.VMEM((S * H,), jnp.float32),
            pltpu.VMEM((H,), jnp.float32),
            pltpu.VMEM((H,), jnp.float32),
            pltpu.VMEM((G, H), jnp.float32),
            pltpu.VMEM((G, H), jnp.float32),
            pltpu.VMEM((G * H,), jnp.float32),
            pltpu.VMEM((G * H,), jnp.float32),
            pltpu.SemaphoreType.DMA,
            pltpu.SemaphoreType.DMA,
            pltpu.SemaphoreType.DMA,
            pltpu.SemaphoreType.DMA,
        ],
    )
    out = f(tok1d, semantic_table, spat, gamma, beta)
    return out.reshape(B, S, H)


# speedup vs baseline: 2.2374x; 1.6039x over previous
import jax
import jax.numpy as jnp
from jax import lax
from jax.experimental import pallas as pl
from jax.experimental.pallas import tpu as pltpu
from jax.experimental.pallas import tpu_sc as plsc

H = 64
S = 200
B = 1024
EPS = 1e-12

NC = 2
NS = 16
NW = NC * NS
SEQ_W = B // NW
L = 16
Q = H // L
S0 = 96
S1 = S - S0


def _rsqrt(x):
    xi = plsc.bitcast(x, jnp.int32)
    yi = jnp.int32(0x5F3759DF) - (xi >> 1)
    y = plsc.bitcast(yi, jnp.float32)
    xh = x * 0.5
    for _ in range(2):
        y = y * (1.5 - xh * y * y)
    return y


def _body(tok_hbm, sem_hbm, spat_hbm, gamma_hbm, beta_hbm, out_hbm,
          idx_v, spat_v, gamma_v, beta_v,
          rows_a, rows_b, out_a, out_b, gsa, gsb, osa, osb):
    wid = lax.axis_index("s") * NC + lax.axis_index("c")
    seq_base = wid * SEQ_W

    pltpu.sync_copy(tok_hbm.at[pl.ds(seq_base, SEQ_W)], idx_v)
    pltpu.sync_copy(spat_hbm, spat_v)
    pltpu.sync_copy(gamma_hbm, gamma_v)
    pltpu.sync_copy(beta_hbm, beta_v)

    def g_start(b, rows_v, sem):
        pltpu.async_copy(sem_hbm.at[idx_v.at[b, pl.ds(0, S0)]],
                         rows_v.at[pl.ds(0, S0)], sem)
        pltpu.async_copy(sem_hbm.at[idx_v.at[b, pl.ds(S0, S1)]],
                         rows_v.at[pl.ds(S0, S1)], sem)

    def g_wait(b, rows_v, sem):
        pltpu.make_async_copy(sem_hbm.at[idx_v.at[b, pl.ds(0, S0)]],
                              rows_v.at[pl.ds(0, S0)], sem).wait()
        pltpu.make_async_copy(sem_hbm.at[idx_v.at[b, pl.ds(S0, S1)]],
                              rows_v.at[pl.ds(S0, S1)], sem).wait()

    def o_dst(b):
        return out_hbm.at[seq_base + b]

    def row(rr, rows_v, out_v, gq, bq):
        sb = rr * H
        c = [rows_v[rr, pl.ds(q * L, L)] + spat_v[pl.ds(sb + q * L, L)]
             for q in range(Q)]
        sv = (c[0] + c[1]) + (c[2] + c[3])
        s2 = (c[0] * c[0] + c[1] * c[1]) + (c[2] * c[2] + c[3] * c[3])
        tot = jnp.full((L,), lax.reduce_sum_p.bind(sv, axes=(0,)), jnp.float32)
        tot2 = jnp.full((L,), lax.reduce_sum_p.bind(s2, axes=(0,)), jnp.float32)
        mean = tot * (1.0 / H)
        var = tot2 * (1.0 / H) - mean * mean
        rstd = _rsqrt(var + EPS)
        for q in range(Q):
            out_v[rr, pl.ds(q * L, L)] = (c[q] - mean) * (rstd * gq[q]) + bq[q]

    def compute(rows_v, out_v):
        gq = [gamma_v[pl.ds(q * L, L)] for q in range(Q)]
        bq = [beta_v[pl.ds(q * L, L)] for q in range(Q)]

        def chunk(cc, c2):
            c0 = cc * L
            for r in range(L):
                row(c0 + r, rows_v, out_v, gq, bq)
            return c2

        lax.fori_loop(0, S // L, chunk, 0)
        for r in range(S - (S // L) * L):
            row((S // L) * L + r, rows_v, out_v, gq, bq)

    g_start(0, rows_a, gsa)

    def blk(kk, carry):
        b0 = 2 * kk
        b1 = b0 + 1
        g_start(b1, rows_b, gsb)
        g_wait(b0, rows_a, gsa)

        @pl.when(kk > 0)
        def _():
            pltpu.make_async_copy(out_a, o_dst(b0 - 2), osa).wait()

        compute(rows_a, out_a)
        pltpu.async_copy(out_a, o_dst(b0), osa)

        @pl.when(kk + 1 < SEQ_W // 2)
        def _():
            g_start(b0 + 2, rows_a, gsa)

        g_wait(b1, rows_b, gsb)

        @pl.when(kk > 0)
        def _():
            pltpu.make_async_copy(out_b, o_dst(b1 - 2), osb).wait()

        compute(rows_b, out_b)
        pltpu.async_copy(out_b, o_dst(b1), osb)
        return carry

    lax.fori_loop(0, SEQ_W // 2, blk, 0)
    pltpu.make_async_copy(out_a, o_dst(SEQ_W - 2), osa).wait()
    pltpu.make_async_copy(out_b, o_dst(SEQ_W - 1), osb).wait()


def kernel(token_idx, semantic_table, spatial_table, gamma, beta):
    tok = token_idx.astype(jnp.int32)
    spat = spatial_table[:S].reshape(S * H)
    mesh = plsc.VectorSubcoreMesh(core_axis_name="c", subcore_axis_name="s")
    f = pl.kernel(
        _body,
        out_type=jax.ShapeDtypeStruct((B, S, H), jnp.float32),
        mesh=mesh,
        compiler_params=pltpu.CompilerParams(
            use_tc_tiling_on_sc=False, needs_layout_passes=False),
        scratch_types=[
            pltpu.VMEM((SEQ_W, S), jnp.int32),
            pltpu.VMEM((S * H,), jnp.float32),
            pltpu.VMEM((H,), jnp.float32),
            pltpu.VMEM((H,), jnp.float32),
            pltpu.VMEM((S, H), jnp.float32),
            pltpu.VMEM((S, H), jnp.float32),
            pltpu.VMEM((S, H), jnp.float32),
            pltpu.VMEM((S, H), jnp.float32),
            pltpu.SemaphoreType.DMA,
            pltpu.SemaphoreType.DMA,
            pltpu.SemaphoreType.DMA,
            pltpu.SemaphoreType.DMA,
        ],
    )
    return f(tok, semantic_table, spat, gamma, beta)
